# Initial kernel scaffold; baseline (speedup 1.0000x reference)
import jax
import jax.numpy as jnp
from jax.experimental import pallas as pl


def kernel(x, head_len):
    raise NotImplementedError("write your pallas kernel here")


# speedup vs baseline: 53.3965x; 53.3965x over previous
import functools

import jax
import jax.numpy as jnp
from jax import lax
from jax.experimental import pallas as pl
from jax.experimental.pallas import tpu as pltpu
from jax.experimental.pallas import tpu_sc as plsc

L = 16
NEG_INF = float("-inf")


def _chunk_reduce(buf, n_vec, carry):
    def body(i, c):
        acc, m1, m2, m3 = c
        v = buf[pl.ds(i * L, L)]
        acc = acc + v
        hi1 = jnp.maximum(m1, v)
        lo1 = jnp.minimum(m1, v)
        hi2 = jnp.maximum(m2, lo1)
        lo2 = jnp.minimum(m2, lo1)
        hi3 = jnp.maximum(m3, lo2)
        return (acc, hi1, hi2, hi3)

    return lax.fori_loop(0, n_vec, body, carry, unroll=8)


def _pop_max(m1, m2, m3):
    t = jnp.max(m1)
    first = plsc.all_reduce_ffs(m1 == t)
    sel = lax.iota(jnp.int32, L) == first
    return t, jnp.where(sel, m2, m1), jnp.where(sel, m3, m2), jnp.where(sel, NEG_INF, m3)


def _make_sc_call(R, N):
    info = plsc.get_sparse_core_info()
    NC, NS = info.num_cores, info.num_subcores
    NW = NC * NS
    assert R % NW == 0
    rows_per_w = R // NW
    NCH = 5
    CH = N // NCH
    assert CH * NCH == N and CH % L == 0 and CH % 8 == 0
    n_chunks = rows_per_w * NCH

    mesh = plsc.VectorSubcoreMesh(core_axis_name="c", subcore_axis_name="s")

    @functools.partial(
        pl.kernel,
        out_type=jax.ShapeDtypeStruct((NW, L), jnp.float32),
        mesh=mesh,
        compiler_params=pltpu.CompilerParams(needs_layout_passes=False),
        scratch_types=[
            pltpu.VMEM((CH,), jnp.float32),
            pltpu.VMEM((CH,), jnp.float32),
            pltpu.VMEM((L,), jnp.float32),
            pltpu.SemaphoreType.DMA,
            pltpu.SemaphoreType.DMA,
        ],
    )
    def sc_call(x_hbm, out_hbm, buf0, buf1, outv, sem0, sem1):
        c = lax.axis_index("c")
        s = lax.axis_index("s")
        wid = s * NC + c
        base = wid * rows_per_w * N
        bufs = (buf0, buf1)
        sems = (sem0, sem1)

        def copy(k):
            return pltpu.make_async_copy(
                x_hbm.at[pl.ds(base + k * CH, CH)], bufs[k % 2], sems[k % 2]
            )

        copy(0).start()
        total = jnp.float32(0.0)
        zeros = jnp.zeros((L,), jnp.float32)
        ninf = jnp.full((L,), NEG_INF)
        carry = (zeros, ninf, ninf, ninf)
        for k in range(n_chunks):
            if k + 1 < n_chunks:
                copy(k + 1).start()
            copy(k).wait()
            carry = _chunk_reduce(bufs[k % 2], CH // L, carry)
            if k % NCH == NCH - 1:
                acc, m1, m2, m3 = carry
                _t1, m1, m2, m3 = _pop_max(m1, m2, m3)
                t2, m1, m2, m3 = _pop_max(m1, m2, m3)
                t3 = jnp.max(m1)
                total = total + (jnp.sum(acc) - t2 - 2.0 * t3)
                carry = (zeros, ninf, ninf, ninf)

        outv[...] = jnp.where(lax.iota(jnp.int32, L) == 0, total, 0.0)
        pltpu.sync_copy(outv, out_hbm.at[wid])

    return sc_call


def kernel(x, head_len):
    del head_len
    R, N = x.shape
    out = _make_sc_call(R, N)(x.reshape(-1))
    return jnp.sum(out)


# speedup vs baseline: 53.3974x; 1.0000x over previous
import functools

import jax
import jax.numpy as jnp
from jax import lax
from jax.experimental import pallas as pl
from jax.experimental.pallas import tpu as pltpu
from jax.experimental.pallas import tpu_sc as plsc

L = 16
NEG_INF = float("-inf")


K = 5


def _insert(state, v):
    acc, m1, m2, m3 = state
    acc = acc + v
    hi1 = jnp.maximum(m1, v)
    lo1 = jnp.minimum(m1, v)
    hi2 = jnp.maximum(m2, lo1)
    lo2 = jnp.minimum(m2, lo1)
    hi3 = jnp.maximum(m3, lo2)
    return (acc, hi1, hi2, hi3)


def _chunk_reduce(buf, n_vec, states):
    assert n_vec % K == 0

    def body(i, sts):
        base = i * (K * L)
        return tuple(
            _insert(sts[j], buf[pl.ds(base + j * L, L)]) for j in range(K)
        )

    return lax.fori_loop(0, n_vec // K, body, states, unroll=2)


def _merge_states(states):
    acc, m1, m2, m3 = states[0]
    for b_acc, b1, b2, b3 in states[1:]:
        acc = acc + b_acc
        _, m1, m2, m3 = _insert((acc, m1, m2, m3), b1)
        hi2 = jnp.maximum(m2, b2)
        lo2 = jnp.minimum(m2, b2)
        m2, m3 = hi2, jnp.maximum(m3, lo2)
        m3 = jnp.maximum(m3, jnp.minimum(m2, b3))
    return acc, m1, m2, m3


def _pop_max(m1, m2, m3):
    t = jnp.max(m1)
    first = plsc.all_reduce_ffs(m1 == t)
    sel = lax.iota(jnp.int32, L) == first
    return t, jnp.where(sel, m2, m1), jnp.where(sel, m3, m2), jnp.where(sel, NEG_INF, m3)


def _make_sc_call(R, N):
    info = plsc.get_sparse_core_info()
    NC, NS = info.num_cores, info.num_subcores
    NW = NC * NS
    assert R % NW == 0
    rows_per_w = R // NW
    NCH = 5
    CH = N // NCH
    assert CH * NCH == N and CH % L == 0 and CH % 8 == 0
    n_chunks = rows_per_w * NCH

    mesh = plsc.VectorSubcoreMesh(core_axis_name="c", subcore_axis_name="s")

    @functools.partial(
        pl.kernel,
        out_type=jax.ShapeDtypeStruct((NW, L), jnp.float32),
        mesh=mesh,
        compiler_params=pltpu.CompilerParams(needs_layout_passes=False),
        scratch_types=[
            pltpu.VMEM((CH,), jnp.float32),
            pltpu.VMEM((CH,), jnp.float32),
            pltpu.VMEM((L,), jnp.float32),
            pltpu.SemaphoreType.DMA,
            pltpu.SemaphoreType.DMA,
        ],
    )
    def sc_call(x_hbm, out_hbm, buf0, buf1, outv, sem0, sem1):
        c = lax.axis_index("c")
        s = lax.axis_index("s")
        wid = s * NC + c
        base = wid * rows_per_w * N
        bufs = (buf0, buf1)
        sems = (sem0, sem1)

        def copy(k):
            return pltpu.make_async_copy(
                x_hbm.at[pl.ds(base + k * CH, CH)], bufs[k % 2], sems[k % 2]
            )

        copy(0).start()
        total = jnp.float32(0.0)
        zeros = jnp.zeros((L,), jnp.float32)
        ninf = jnp.full((L,), NEG_INF)
        fresh = tuple((zeros, ninf, ninf, ninf) for _ in range(K))
        states = fresh
        for k in range(n_chunks):
            if k + 1 < n_chunks:
                copy(k + 1).start()
            copy(k).wait()
            states = _chunk_reduce(bufs[k % 2], CH // L, states)
            if k % NCH == NCH - 1:
                acc, m1, m2, m3 = _merge_states(states)
                _t1, m1, m2, m3 = _pop_max(m1, m2, m3)
                t2, m1, m2, m3 = _pop_max(m1, m2, m3)
                t3 = jnp.max(m1)
                total = total + (jnp.sum(acc) - t2 - 2.0 * t3)
                states = fresh

        outv[...] = jnp.where(lax.iota(jnp.int32, L) == 0, total, 0.0)
        pltpu.sync_copy(outv, out_hbm.at[wid])

    return sc_call


def kernel(x, head_len):
    del head_len
    R, N = x.shape
    out = _make_sc_call(R, N)(x.reshape(-1))
    return jnp.sum(out)


# speedup vs baseline: 88.8386x; 1.6637x over previous
import functools

import jax
import jax.numpy as jnp
from jax import lax
from jax.experimental import pallas as pl
from jax.experimental.pallas import tpu as pltpu
from jax.experimental.pallas import tpu_sc as plsc

L = 16
RB = 8
NEG_INF = float("-inf")


def _insert(state, v):
    acc, m1, m2, m3 = state
    acc = acc + v
    hi1 = jnp.maximum(m1, v)
    lo1 = jnp.minimum(m1, v)
    hi2 = jnp.maximum(m2, lo1)
    lo2 = jnp.minimum(m2, lo1)
    hi3 = jnp.maximum(m3, lo2)
    return (acc, hi1, hi2, hi3)


def _merge_states(a, b):
    acc, m1, m2, m3 = a
    b_acc, b1, b2, b3 = b
    acc = acc + b_acc
    _, m1, m2, m3 = _insert((acc, m1, m2, m3), b1)
    hi2 = jnp.maximum(m2, b2)
    lo2 = jnp.minimum(m2, b2)
    m2, m3 = hi2, jnp.maximum(m3, lo2)
    m3 = jnp.maximum(m3, jnp.minimum(m2, b3))
    return acc, m1, m2, m3


def _chunk_reduce(buf, n_vec, states):
    def body(i, sts):
        col = i * L
        return tuple(
            _insert(sts[r], buf[r, pl.ds(col, L)]) for r in range(RB)
        )

    return lax.fori_loop(0, n_vec, body, states, unroll=2)


def _pop_max(m1, m2, m3):
    t = jnp.max(m1)
    first = plsc.all_reduce_ffs(m1 == t)
    sel = lax.iota(jnp.int32, L) == first
    return t, jnp.where(sel, m2, m1), jnp.where(sel, m3, m2), jnp.where(sel, NEG_INF, m3)


def _make_sc_call(R, N):
    info = plsc.get_sparse_core_info()
    NC, NS = info.num_cores, info.num_subcores
    n_blocks = R // RB
    assert n_blocks == NC * NS // 2
    n_tiles = (N // 128) // 2 * 2
    HALF = n_tiles // 2 * 128
    TAIL0, TAILW = 2 * HALF, N - 2 * HALF
    NCH = 10
    CW = HALF // NCH
    assert CW % 128 == 0 and CW * NCH == HALF and TAILW % L == 0

    mesh = plsc.VectorSubcoreMesh(core_axis_name="c", subcore_axis_name="s")

    @functools.partial(
        pl.kernel,
        out_type=jax.ShapeDtypeStruct((n_blocks, L), jnp.float32),
        mesh=mesh,
        compiler_params=pltpu.CompilerParams(needs_layout_passes=False),
        scratch_types=[
            pltpu.VMEM((RB, CW), jnp.float32),
            pltpu.VMEM((RB, CW), jnp.float32),
            pltpu.VMEM((RB, TAILW), jnp.float32),
            pltpu.VMEM((RB, 4, L), jnp.float32),
            pltpu.VMEM((RB, 4, L), jnp.float32),
            pltpu.VMEM((L,), jnp.float32),
            pltpu.VMEM_SHARED((NS, RB, 4, L), jnp.float32),
            pltpu.SemaphoreType.DMA,
            pltpu.SemaphoreType.DMA,
            pltpu.SemaphoreType.DMA,
        ],
    )
    def sc_call(
        x_hbm, out_hbm, buf0, buf1, tailbuf, statebuf, partnerbuf, outv,
        shared, sem0, sem1, semt,
    ):
        c = lax.axis_index("c")
        s = lax.axis_index("s")
        rb = c * RB + s // 2
        h = s % 2
        r0 = rb * RB
        col0 = h * HALF
        bufs = (buf0, buf1)
        sems = (sem0, sem1)

        def copy(k):
            return pltpu.make_async_copy(
                x_hbm.at[pl.ds(r0, RB), pl.ds(col0 + k * CW, CW)],
                bufs[k % 2],
                sems[k % 2],
            )

        tail_copy = pltpu.make_async_copy(
            x_hbm.at[pl.ds(r0, RB), pl.ds(TAIL0, TAILW)], tailbuf, semt
        )
        copy(0).start()
        tail_copy.start()

        zeros = jnp.zeros((L,), jnp.float32)
        ninf = jnp.full((L,), NEG_INF)
        states = tuple((zeros, ninf, ninf, ninf) for _ in range(RB))
        for k in range(NCH):
            if k + 1 < NCH:
                copy(k + 1).start()
            copy(k).wait()
            states = _chunk_reduce(bufs[k % 2], CW // L, states)

        tail_copy.wait()
        states = list(states)
        for r in range(RB):
            mine = (r // (RB // 2)) == h
            st = states[r]
            for j in range(TAILW // L):
                v = tailbuf[r, pl.ds(j * L, L)]
                acc, m1, m2, m3 = st
                acc = acc + jnp.where(mine, v, 0.0)
                st = _insert((acc, m1, m2, m3), jnp.where(mine, v, NEG_INF))
                st = (acc, st[1], st[2], st[3])
            states[r] = st

        for r in range(RB):
            acc, m1, m2, m3 = states[r]
            statebuf[r, 0] = acc
            statebuf[r, 1] = m1
            statebuf[r, 2] = m2
            statebuf[r, 3] = m3

        @pl.when(h == 1)
        def _publish():
            pltpu.sync_copy(statebuf, shared.at[s])

        plsc.subcore_barrier()

        @pl.when(h == 0)
        def _reduce():
            pltpu.sync_copy(shared.at[s + 1], partnerbuf)
            total = jnp.float32(0.0)
            for r in range(RB):
                mine = (
                    statebuf[r, 0], statebuf[r, 1], statebuf[r, 2], statebuf[r, 3]
                )
                other = (
                    partnerbuf[r, 0], partnerbuf[r, 1], partnerbuf[r, 2],
                    partnerbuf[r, 3],
                )
                acc, m1, m2, m3 = _merge_states(mine, other)
                _t1, m1, m2, m3 = _pop_max(m1, m2, m3)
                t2, m1, m2, m3 = _pop_max(m1, m2, m3)
                t3 = jnp.max(m1)
                total = total + (jnp.sum(acc) - t2 - 2.0 * t3)
            outv[...] = jnp.where(lax.iota(jnp.int32, L) == 0, total, 0.0)
            pltpu.sync_copy(outv, out_hbm.at[rb])

    return sc_call


def kernel(x, head_len):
    del head_len
    R, N = x.shape
    out = _make_sc_call(R, N)(x)
    return jnp.sum(out)


# speedup vs baseline: 154.2239x; 1.7360x over previous
import functools

import jax
import jax.numpy as jnp
from jax import lax
from jax.experimental import pallas as pl
from jax.experimental.pallas import tpu as pltpu
from jax.experimental.pallas import tpu_sc as plsc

L = 16
NG = 8
NEG_INF = float("-inf")


def _insert(state, v, vs=None):
    acc, m1, m2, m3 = state
    acc = acc + (v if vs is None else vs)
    hi1 = jnp.maximum(m1, v)
    lo1 = jnp.minimum(m1, v)
    hi2 = jnp.maximum(m2, lo1)
    lo2 = jnp.minimum(m2, lo1)
    hi3 = jnp.maximum(m3, lo2)
    return (acc, hi1, hi2, hi3)


def _merge_states(a, b):
    acc, m1, m2, m3 = a
    b_acc, b1, b2, b3 = b
    acc = acc + b_acc
    _, m1, m2, m3 = _insert((acc, m1, m2, m3), b1, vs=jnp.zeros_like(b1))
    hi2 = jnp.maximum(m2, b2)
    lo2 = jnp.minimum(m2, b2)
    m2, m3 = hi2, jnp.maximum(m3, lo2)
    m3 = jnp.maximum(m3, jnp.minimum(m2, b3))
    return acc, m1, m2, m3


def _make_sc_call(N, R):
    info = plsc.get_sparse_core_info()
    NC, NS = info.num_cores, info.num_subcores
    NW = NC * NS
    assert R == NG * L
    n_tiles = N // 8
    main_tiles = n_tiles // NW * NW
    MAIN = main_tiles * 8
    STRIPE = MAIN // NW
    tail_tiles = n_tiles - main_tiles
    assert tail_tiles <= NW and (N - MAIN) == tail_tiles * 8
    NCH = 10
    CJ = STRIPE // NCH
    assert CJ % 8 == 0 and CJ * NCH == STRIPE

    mesh = plsc.VectorSubcoreMesh(core_axis_name="c", subcore_axis_name="s")

    @functools.partial(
        pl.kernel,
        out_type=jax.ShapeDtypeStruct((NC, NG, 4, L), jnp.float32),
        mesh=mesh,
        compiler_params=pltpu.CompilerParams(needs_layout_passes=False),
        scratch_types=[
            pltpu.VMEM((CJ, R), jnp.float32),
            pltpu.VMEM((CJ, R), jnp.float32),
            pltpu.VMEM((8, R), jnp.float32),
            pltpu.VMEM((NG, 4, L), jnp.float32),
            pltpu.VMEM((NS, 4, L), jnp.float32),
            pltpu.VMEM((4, L), jnp.float32),
            pltpu.VMEM_SHARED((NG, NS, 4, L), jnp.float32),
            pltpu.SemaphoreType.DMA,
            pltpu.SemaphoreType.DMA,
            pltpu.SemaphoreType.DMA,
        ],
    )
    def sc_call(
        xt_hbm, out_hbm, buf0, buf1, tailbuf, statebuf, gatherbuf, mergebuf,
        shared, sem0, sem1, semt,
    ):
        c = lax.axis_index("c")
        s = lax.axis_index("s")
        w = c * NS + s
        j0 = w * STRIPE
        bufs = (buf0, buf1)
        sems = (sem0, sem1)

        def copy(k):
            return pltpu.make_async_copy(
                xt_hbm.at[pl.ds(j0 + k * CJ, CJ)], bufs[k % 2], sems[k % 2]
            )

        tw = jnp.where(w < tail_tiles, w, w - tail_tiles)
        tail_copy = pltpu.make_async_copy(
            xt_hbm.at[pl.ds(MAIN + 8 * tw, 8)], tailbuf, semt
        )
        copy(0).start()
        tail_copy.start()

        zeros = jnp.zeros((L,), jnp.float32)
        ninf = jnp.full((L,), NEG_INF)
        states = tuple((zeros, ninf, ninf, ninf) for _ in range(NG))

        def chunk_body(jj, sts):
            buf = bufs_cur[0]
            return tuple(
                _insert(sts[g], buf[jj, pl.ds(g * L, L)]) for g in range(NG)
            )

        for k in range(NCH):
            if k + 1 < NCH:
                copy(k + 1).start()
            copy(k).wait()
            bufs_cur = (bufs[k % 2],)
            states = lax.fori_loop(0, CJ, chunk_body, states, unroll=2)

        tail_copy.wait()
        valid = w < tail_tiles
        states = list(states)
        for jj in range(8):
            for g in range(NG):
                v = tailbuf[jj, pl.ds(g * L, L)]
                vt = jnp.where(valid, v, NEG_INF)
                vs = jnp.where(valid, v, 0.0)
                states[g] = _insert(states[g], vt, vs=vs)

        for g in range(NG):
            acc, m1, m2, m3 = states[g]
            statebuf[g, 0] = acc
            statebuf[g, 1] = m1
            statebuf[g, 2] = m2
            statebuf[g, 3] = m3
            pltpu.sync_copy(statebuf.at[g], shared.at[g, s])

        plsc.subcore_barrier()

        @pl.when(s < NG)
        def _reduce():
            pltpu.sync_copy(shared.at[s], gatherbuf)
            acc = gatherbuf[0, 0]
            m1 = gatherbuf[0, 1]
            m2 = gatherbuf[0, 2]
            m3 = gatherbuf[0, 3]
            st = (acc, m1, m2, m3)
            for w2 in range(1, NS):
                other = (
                    gatherbuf[w2, 0], gatherbuf[w2, 1],
                    gatherbuf[w2, 2], gatherbuf[w2, 3],
                )
                st = _merge_states(st, other)
            mergebuf[0] = st[0]
            mergebuf[1] = st[1]
            mergebuf[2] = st[2]
            mergebuf[3] = st[3]
            pltpu.sync_copy(mergebuf, out_hbm.at[c, s])

    return sc_call


def kernel(x, head_len):
    del head_len
    R, N = x.shape
    out = _make_sc_call(N, R)(x.T)
    a = (out[0, :, 0], out[0, :, 1], out[0, :, 2], out[0, :, 3])
    b = (out[1, :, 0], out[1, :, 1], out[1, :, 2], out[1, :, 3])
    acc, _m1, m2, m3 = _merge_states(a, b)
    return jnp.sum(acc - m2 - 2.0 * m3)
